# Initial kernel scaffold; baseline (speedup 1.0000x reference)
import jax
import jax.numpy as jnp
from jax.experimental import pallas as pl


def kernel(x, edge_index, W1, a_src1, a_dst1, b1, W2, a_src2, a_dst2, b2):
    raise NotImplementedError("write your pallas kernel here")


# speedup vs baseline: 14.2406x; 14.2406x over previous
import functools

import jax
import jax.numpy as jnp
from jax import lax
from jax.experimental import pallas as pl
from jax.experimental.pallas import tpu as pltpu
from jax.experimental.pallas import tpu_sc as plsc

N = 10000
E = 320000
F_IN = 128
HID = 32
HEADS = 8
NUM_CLASS = 16

R = 1000
K1 = 80
K2 = 80
NT = 16
NSC = 2
SLAB = 632
ZCHUNKS = (80, 80, 80, 80, 80, 80, 80, 72)
EPT1 = E // NT
EPT2 = E // (NSC * NT)


def _leaky(v):
    return jnp.where(v > 0, v, 0.2 * v)


def _dense1_body(x_ref, w1_ref, asf_ref, adf_ref, hh_ref, s_ref, d_ref):
    h = jnp.dot(x_ref[...], w1_ref[...], preferred_element_type=jnp.float32)
    row = lax.broadcasted_iota(jnp.int32, (HEADS * HID, 16), 0) // HID
    col = lax.broadcasted_iota(jnp.int32, (HEADS * HID, 16), 1)
    m = (row == col).astype(jnp.float32)
    s_ref[...] = jnp.dot(h * asf_ref[...], m, preferred_element_type=jnp.float32)
    d_ref[...] = jnp.dot(h * adf_ref[...], m, preferred_element_type=jnp.float32)
    hh_ref[0] = h[:, :128]
    hh_ref[1] = h[:, 128:]


def _dense1(x, W1, asf, adf):
    return pl.pallas_call(
        _dense1_body,
        grid=(N // R,),
        in_specs=[
            pl.BlockSpec((R, F_IN), lambda i: (i, 0)),
            pl.BlockSpec((F_IN, HEADS * HID), lambda i: (0, 0)),
            pl.BlockSpec((1, HEADS * HID), lambda i: (0, 0)),
            pl.BlockSpec((1, HEADS * HID), lambda i: (0, 0)),
        ],
        out_specs=[
            pl.BlockSpec((2, R, 128), lambda i: (0, i, 0)),
            pl.BlockSpec((R, 16), lambda i: (i, 0)),
            pl.BlockSpec((R, 16), lambda i: (i, 0)),
        ],
        out_shape=[
            jax.ShapeDtypeStruct((2, N, 128), jnp.float32),
            jax.ShapeDtypeStruct((N, 16), jnp.float32),
            jax.ShapeDtypeStruct((N, 16), jnp.float32),
        ],
    )(x, W1, asf, adf)


def _edge0_body(src_hbm, dst_hbm, sh_hbm, dh_hbm,
                wtab_hbm, den_hbm,
                den_sh, srcv, srcv4, dstv, dstv4, wbuf, sflat, dflat, wgrp,
                sem):
    c = lax.axis_index("c")
    t = lax.axis_index("s")

    pltpu.sync_copy(sh_hbm.at[c], sflat)
    pltpu.sync_copy(dh_hbm.at[c], dflat)

    def zero_body(e, carry):
        wbuf[e, pl.ds(0, 16)] = jnp.zeros((16,), jnp.float32)
        return carry

    lax.fori_loop(0, K1, zero_body, 0)
    for g in range(21):
        wgrp[pl.ds(16 * g, 16)] = jnp.zeros((16,), jnp.float32)
    row0 = jnp.minimum(t * SLAB, N - SLAB)
    off = 0
    for nr in ZCHUNKS:
        pltpu.sync_copy(wbuf.at[pl.ds(0, nr)],
                        den_sh.at[pl.ds(row0 + off, nr)])
        off += nr
    plsc.subcore_barrier()

    lanes = lax.iota(jnp.int32, 16)
    zb0 = jnp.where(lanes < 4, 80 * lanes, 320)
    zbm = (lanes < 4).astype(jnp.int32)

    def body(j, carry):
        base = t * EPT1 + j * K1
        pltpu.sync_copy(src_hbm.at[pl.ds(base, K1)], srcv)
        pltpu.sync_copy(dst_hbm.at[pl.ds(base, K1)], dstv)
        for v in range(K1 // 16):
            srcv4[pl.ds(16 * v, 16)] = srcv[pl.ds(16 * v, 16)] * 4
            dstv4[pl.ds(16 * v, 16)] = dstv[pl.ds(16 * v, 16)] * 4
        for v in range(K1 // 16):
            s4 = srcv4[pl.ds(16 * v, 16)]
            d4 = dstv4[pl.ds(16 * v, 16)]
            for h in range(4):
                sv = plsc.load_gather(sflat, [s4 + h])
                dv = plsc.load_gather(dflat, [d4 + h])
                wgrp[pl.ds(80 * h + 16 * v, 16)] = jnp.exp(_leaky(sv + dv))
        for e in range(K1):
            wbuf[e, pl.ds(0, 16)] = plsc.load_gather(wgrp, [zb0 + zbm * e])
        pltpu.sync_copy(wbuf, den_sh.at[dstv], add=True)
        pltpu.sync_copy(wbuf, wtab_hbm.at[c, pl.ds(base, K1)])
        return carry

    lax.fori_loop(0, EPT1 // K1, body, 0)
    plsc.subcore_barrier()
    pltpu.sync_copy(den_sh.at[pl.ds(row0, SLAB)],
                    den_hbm.at[c, pl.ds(row0, SLAB)])


@functools.lru_cache(maxsize=1)
def _edge0():
    return pl.kernel(
        _edge0_body,
        out_type=[
            jax.ShapeDtypeStruct((2, E, 16), jnp.float32),
            jax.ShapeDtypeStruct((2, N, 16), jnp.float32),
        ],
        mesh=plsc.VectorSubcoreMesh(core_axis_name="c", subcore_axis_name="s",
                                    num_cores=NSC, num_subcores=NT),
        compiler_params=pltpu.CompilerParams(needs_layout_passes=False),
        scratch_types=[
            pltpu.VMEM_SHARED((N, 16), jnp.float32),
            pltpu.VMEM((K1,), jnp.int32),
            pltpu.VMEM((K1,), jnp.int32),
            pltpu.VMEM((K1,), jnp.int32),
            pltpu.VMEM((K1,), jnp.int32),
            pltpu.VMEM((K1, 16), jnp.float32),
            pltpu.VMEM((4 * N,), jnp.float32),
            pltpu.VMEM((4 * N,), jnp.float32),
            pltpu.VMEM((336,), jnp.float32),
            pltpu.SemaphoreType.DMA,
        ],
    )


def _edge1_body(src_hbm, dst_hbm, h01_hbm, wtabf_hbm,
                num_hbm,
                num_sh, srcv, srcv2, dstv, hbuf, wflat, sem):
    c = lax.axis_index("c")
    t = lax.axis_index("s")
    cn = c * N

    def zero_body(e, carry):
        for g in range(8):
            hbuf[e, pl.ds(16 * g, 16)] = jnp.zeros((16,), jnp.float32)
        return carry

    lax.fori_loop(0, K1, zero_body, 0)
    row0 = jnp.minimum(t * SLAB, N - SLAB)
    off = 0
    for nr in ZCHUNKS:
        pltpu.sync_copy(hbuf.at[pl.ds(0, nr)],
                        num_sh.at[pl.ds(row0 + off, nr)])
        off += nr
    plsc.subcore_barrier()

    def body(j, carry):
        base = t * EPT1 + j * K1
        pltpu.sync_copy(src_hbm.at[pl.ds(base, K1)], srcv)
        pltpu.sync_copy(dst_hbm.at[pl.ds(base, K1)], dstv)
        pltpu.sync_copy(wtabf_hbm.at[c, pl.ds(16 * base, 16 * K1)], wflat)
        for v in range(K1 // 16):
            srcv2[pl.ds(16 * v, 16)] = srcv[pl.ds(16 * v, 16)] + cn
        pltpu.async_copy(h01_hbm.at[srcv2], hbuf, sem).wait()
        for e in range(K1):
            for h in range(4):
                ws = plsc.load_gather(wflat, [jnp.full((16,), 16 * e + h,
                                                       jnp.int32)])
                for g in range(2):
                    o = 32 * h + 16 * g
                    hbuf[e, pl.ds(o, 16)] = hbuf[e, pl.ds(o, 16)] * ws
        pltpu.sync_copy(hbuf, num_sh.at[dstv], add=True)
        return carry

    lax.fori_loop(0, EPT1 // K1, body, 0)
    plsc.subcore_barrier()
    pltpu.sync_copy(num_sh.at[pl.ds(row0, SLAB)],
                    num_hbm.at[c, pl.ds(row0, SLAB)])


@functools.lru_cache(maxsize=1)
def _edge1():
    return pl.kernel(
        _edge1_body,
        out_type=jax.ShapeDtypeStruct((2, N, 128), jnp.float32),
        mesh=plsc.VectorSubcoreMesh(core_axis_name="c", subcore_axis_name="s",
                                    num_cores=NSC, num_subcores=NT),
        compiler_params=pltpu.CompilerParams(needs_layout_passes=False),
        scratch_types=[
            pltpu.VMEM_SHARED((N, 128), jnp.float32),
            pltpu.VMEM((K1,), jnp.int32),
            pltpu.VMEM((K1,), jnp.int32),
            pltpu.VMEM((K1,), jnp.int32),
            pltpu.VMEM((K1, 128), jnp.float32),
            pltpu.VMEM((16 * K1,), jnp.float32),
            pltpu.SemaphoreType.DMA,
        ],
    )


def _dense2_body(num_ref, den_ref, hh_ref, s_ref, d_ref, b1_ref, w2_ref,
                 a2s_ref, a2d_ref, t2_ref, h2s_ref, s2o_ref, d2o_ref):
    s = s_ref[...][:, :8]
    d = d_ref[...][:, :8]
    wself = jnp.exp(_leaky(s + d))
    den8 = jnp.concatenate([den_ref[0][:, :4], den_ref[1][:, :4]], axis=1)
    dinv = 1.0 / (den8 + wself + 1e-16)
    rowi = lax.broadcasted_iota(jnp.int32, (8, 128), 0)
    coli = lax.broadcasted_iota(jnp.int32, (8, 128), 1) // HID
    h2 = jnp.zeros((t2_ref.shape[0], 16), jnp.float32)
    for c in range(2):
        ec = (coli == (rowi - 4 * c)).astype(jnp.float32)
        wexp = jnp.dot(wself, ec, preferred_element_type=jnp.float32)
        dexp = jnp.dot(dinv, ec, preferred_element_type=jnp.float32)
        out1 = (num_ref[c] + hh_ref[c] * wexp) * dexp \
            + b1_ref[...][:, 128 * c:128 * (c + 1)]
        z = jnp.where(out1 > 0, out1, jnp.exp(out1) - 1.0)
        h2 = h2 + jnp.dot(z, w2_ref[...][128 * c:128 * (c + 1), :],
                          preferred_element_type=jnp.float32)
    ri = lax.broadcasted_iota(jnp.int32, (16, 32), 0)
    ci = lax.broadcasted_iota(jnp.int32, (16, 32), 1)
    aug = (ri == ci).astype(jnp.float32) \
        + a2s_ref[...] * (ci == 16).astype(jnp.float32) \
        + a2d_ref[...] * (ci == 17).astype(jnp.float32)
    t2 = jnp.dot(h2, aug, preferred_element_type=jnp.float32)
    t2_ref[...] = t2
    h2s_ref[0] = h2[:, :8]
    h2s_ref[1] = h2[:, 8:]
    s2o_ref[...] = t2[:, 16:17]
    d2o_ref[...] = t2[:, 17:18]


def _dense2(num, den, hh, s16, d16, b1, W2, a2s, a2d):
    return pl.pallas_call(
        _dense2_body,
        grid=(N // R,),
        in_specs=[
            pl.BlockSpec((2, R, 128), lambda i: (0, i, 0)),
            pl.BlockSpec((2, R, 16), lambda i: (0, i, 0)),
            pl.BlockSpec((2, R, 128), lambda i: (0, i, 0)),
            pl.BlockSpec((R, 16), lambda i: (i, 0)),
            pl.BlockSpec((R, 16), lambda i: (i, 0)),
            pl.BlockSpec((1, 256), lambda i: (0, 0)),
            pl.BlockSpec((256, 16), lambda i: (0, 0)),
            pl.BlockSpec((16, 1), lambda i: (0, 0)),
            pl.BlockSpec((16, 1), lambda i: (0, 0)),
        ],
        out_specs=[
            pl.BlockSpec((R, 32), lambda i: (i, 0)),
            pl.BlockSpec((2, R, 8), lambda i: (0, i, 0)),
            pl.BlockSpec((R, 1), lambda i: (i, 0)),
            pl.BlockSpec((R, 1), lambda i: (i, 0)),
        ],
        out_shape=[
            jax.ShapeDtypeStruct((N, 32), jnp.float32),
            jax.ShapeDtypeStruct((2, N, 8), jnp.float32),
            jax.ShapeDtypeStruct((N, 1), jnp.float32),
            jax.ShapeDtypeStruct((N, 1), jnp.float32),
        ],
    )(num, den, hh, s16, d16, b1, W2, a2s, a2d)


def _edge2_body(src_hbm, dst_hbm, h2s_hbm, s2_hbm, d2_hbm, acc_hbm,
                acc_sh, srcv, dstv, srcv8, mbuf, w2buf, h2pad, s2v, d2v, sem):
    c = lax.axis_index("c")
    t = lax.axis_index("s")

    pltpu.sync_copy(h2s_hbm.at[c], h2pad.at[pl.ds(0, 8 * N)])
    pltpu.sync_copy(s2_hbm, s2v)
    pltpu.sync_copy(d2_hbm, d2v)
    h2pad[pl.ds(8 * N, 16)] = jnp.zeros((16,), jnp.float32)

    def zero_body(e, carry):
        mbuf[e, pl.ds(0, 16)] = jnp.zeros((16,), jnp.float32)
        return carry

    lax.fori_loop(0, K2, zero_body, 0)
    row0 = jnp.minimum(t * SLAB, N - SLAB)
    off = 0
    for nr in ZCHUNKS:
        pltpu.sync_copy(mbuf.at[pl.ds(0, nr)],
                        acc_sh.at[pl.ds(row0 + off, nr)])
        off += nr
    plsc.subcore_barrier()

    lanes = lax.iota(jnp.int32, 16)
    lane8 = lanes < 8
    mask8 = (lanes == 8).astype(jnp.float32)

    def body(j, carry):
        base = t * EPT1 + j * K2
        pltpu.sync_copy(src_hbm.at[pl.ds(base, K2)], srcv)
        pltpu.sync_copy(dst_hbm.at[pl.ds(base, K2)], dstv)
        for v in range(K2 // 16):
            sl = srcv[pl.ds(16 * v, 16)]
            srcv8[pl.ds(16 * v, 16)] = sl * 8
            sv = plsc.load_gather(s2v, [sl])
            dv = plsc.load_gather(d2v, [dstv[pl.ds(16 * v, 16)]])
            w2buf[pl.ds(16 * v, 16)] = jnp.exp(_leaky(sv + dv))
        for e in range(K2):
            esplat = jnp.full((16,), e, jnp.int32)
            bs = plsc.load_gather(srcv8, [esplat])
            idx = jnp.where(lane8, bs + lanes, 8 * N)
            mrow = plsc.load_gather(h2pad, [idx])
            ws = plsc.load_gather(w2buf, [esplat])
            mbuf[e, pl.ds(0, 16)] = (mrow + mask8) * ws
        pltpu.sync_copy(mbuf, acc_sh.at[dstv], add=True)
        return carry

    lax.fori_loop(0, EPT1 // K2, body, 0)
    plsc.subcore_barrier()
    pltpu.sync_copy(acc_sh.at[pl.ds(row0, SLAB)],
                    acc_hbm.at[c, pl.ds(row0, SLAB)])


@functools.lru_cache(maxsize=1)
def _edge2():
    return pl.kernel(
        _edge2_body,
        out_type=jax.ShapeDtypeStruct((2, N, 16), jnp.float32),
        mesh=plsc.VectorSubcoreMesh(core_axis_name="c", subcore_axis_name="s",
                                    num_cores=NSC, num_subcores=NT),
        compiler_params=pltpu.CompilerParams(needs_layout_passes=False),
        scratch_types=[
            pltpu.VMEM_SHARED((N, 16), jnp.float32),
            pltpu.VMEM((K2,), jnp.int32),
            pltpu.VMEM((K2,), jnp.int32),
            pltpu.VMEM((K2,), jnp.int32),
            pltpu.VMEM((K2, 16), jnp.float32),
            pltpu.VMEM((K2,), jnp.float32),
            pltpu.VMEM((8 * N + 16,), jnp.float32),
            pltpu.VMEM((N,), jnp.float32),
            pltpu.VMEM((N,), jnp.float32),
            pltpu.SemaphoreType.DMA,
        ],
    )


def _final_body(acc_ref, t2_ref, b2_ref, out_ref):
    t2 = t2_ref[...]
    h2 = t2[:, :16]
    v = t2[:, 16:17] + t2[:, 17:18]
    w2 = jnp.exp(_leaky(v))
    num2 = jnp.concatenate([acc_ref[0][:, :8], acc_ref[1][:, :8]],
                           axis=1) + h2 * w2
    den2 = acc_ref[0][:, 8:9] + w2 + 1e-16
    o = num2 / den2 + b2_ref[...]
    m = jnp.max(o, axis=1, keepdims=True)
    out_ref[...] = (o - m) - jnp.log(jnp.sum(jnp.exp(o - m), axis=1,
                                             keepdims=True))


def _final(acc, t2, b2):
    return pl.pallas_call(
        _final_body,
        grid=(N // R,),
        in_specs=[
            pl.BlockSpec((2, R, 16), lambda i: (0, i, 0)),
            pl.BlockSpec((R, 32), lambda i: (i, 0)),
            pl.BlockSpec((1, 16), lambda i: (0, 0)),
        ],
        out_specs=pl.BlockSpec((R, 16), lambda i: (i, 0)),
        out_shape=jax.ShapeDtypeStruct((N, 16), jnp.float32),
    )(acc, t2, b2)


def kernel(x, edge_index, W1, a_src1, a_dst1, b1, W2, a_src2, a_dst2, b2):
    src = edge_index[0]
    dst = edge_index[1]
    asf = a_src1.reshape(1, HEADS * HID)
    adf = a_dst1.reshape(1, HEADS * HID)
    hh, s16, d16 = _dense1(x, W1, asf, adf)
    h01 = hh.reshape(2 * N, 128)
    shf = jnp.stack([s16[:, 0:4], s16[:, 4:8]]).reshape(2, 4 * N)
    dhf = jnp.stack([d16[:, 0:4], d16[:, 4:8]]).reshape(2, 4 * N)
    wtab, den = _edge0()(src, dst, shf, dhf)
    num = _edge1()(src, dst, h01, wtab.reshape(2, 16 * E))
    t2, h2s, s2o, d2o = _dense2(num, den, hh, s16, d16, b1.reshape(1, 256),
                                W2, a_src2.reshape(16, 1),
                                a_dst2.reshape(16, 1))
    acc = _edge2()(src, dst, h2s.reshape(2, 8 * N), s2o.reshape(N),
                   d2o.reshape(N))
    return _final(acc, t2, b2.reshape(1, 16))


# speedup vs baseline: 18.8438x; 1.3232x over previous
import functools

import jax
import jax.numpy as jnp
from jax import lax
from jax.experimental import pallas as pl
from jax.experimental.pallas import tpu as pltpu
from jax.experimental.pallas import tpu_sc as plsc

N = 10000
E = 320000
F_IN = 128
HID = 32
HEADS = 8
NUM_CLASS = 16

R = 1000
K1 = 80
K2 = 80
NT = 16
NSC = 2
SLAB = 632
ZCHUNKS = (80, 80, 80, 80, 80, 80, 80, 72)
EPT1 = E // NT
EPT2 = E // (NSC * NT)


def _leaky(v):
    return jnp.where(v > 0, v, 0.2 * v)


def _dense1_body(x_ref, w1_ref, asf_ref, adf_ref, hh_ref, s_ref, d_ref):
    h = jnp.dot(x_ref[...], w1_ref[...], preferred_element_type=jnp.float32)
    row = lax.broadcasted_iota(jnp.int32, (HEADS * HID, 16), 0) // HID
    col = lax.broadcasted_iota(jnp.int32, (HEADS * HID, 16), 1)
    m = (row == col).astype(jnp.float32)
    s_ref[...] = jnp.dot(h * asf_ref[...], m, preferred_element_type=jnp.float32)
    d_ref[...] = jnp.dot(h * adf_ref[...], m, preferred_element_type=jnp.float32)
    hh_ref[0] = h[:, :128]
    hh_ref[1] = h[:, 128:]


def _dense1(x, W1, asf, adf):
    return pl.pallas_call(
        _dense1_body,
        grid=(N // R,),
        in_specs=[
            pl.BlockSpec((R, F_IN), lambda i: (i, 0)),
            pl.BlockSpec((F_IN, HEADS * HID), lambda i: (0, 0)),
            pl.BlockSpec((1, HEADS * HID), lambda i: (0, 0)),
            pl.BlockSpec((1, HEADS * HID), lambda i: (0, 0)),
        ],
        out_specs=[
            pl.BlockSpec((2, R, 128), lambda i: (0, i, 0)),
            pl.BlockSpec((R, 16), lambda i: (i, 0)),
            pl.BlockSpec((R, 16), lambda i: (i, 0)),
        ],
        out_shape=[
            jax.ShapeDtypeStruct((2, N, 128), jnp.float32),
            jax.ShapeDtypeStruct((N, 16), jnp.float32),
            jax.ShapeDtypeStruct((N, 16), jnp.float32),
        ],
    )(x, W1, asf, adf)


def _edge0_body(src_hbm, dst_hbm, sh_hbm, dh_hbm,
                wtab_hbm, den_hbm,
                den_sh, srcv, srcv4, dstv, dstv4, wbuf, sflat, dflat, wgrp,
                sem):
    c = lax.axis_index("c")
    t = lax.axis_index("s")

    pltpu.sync_copy(sh_hbm.at[c], sflat)
    pltpu.sync_copy(dh_hbm.at[c], dflat)

    def zero_body(e, carry):
        wbuf[e, pl.ds(0, 16)] = jnp.zeros((16,), jnp.float32)
        return carry

    lax.fori_loop(0, K1, zero_body, 0)
    for g in range(21):
        wgrp[pl.ds(16 * g, 16)] = jnp.zeros((16,), jnp.float32)
    row0 = jnp.minimum(t * SLAB, N - SLAB)
    off = 0
    for nr in ZCHUNKS:
        pltpu.sync_copy(wbuf.at[pl.ds(0, nr)],
                        den_sh.at[pl.ds(row0 + off, nr)])
        off += nr
    plsc.subcore_barrier()

    lanes = lax.iota(jnp.int32, 16)
    zb0 = jnp.where(lanes < 4, 80 * lanes, 320)
    zbm = (lanes < 4).astype(jnp.int32)

    def body(j, carry):
        base = t * EPT1 + j * K1
        pltpu.sync_copy(src_hbm.at[pl.ds(base, K1)], srcv)
        pltpu.sync_copy(dst_hbm.at[pl.ds(base, K1)], dstv)
        for v in range(K1 // 16):
            srcv4[pl.ds(16 * v, 16)] = srcv[pl.ds(16 * v, 16)] * 4
            dstv4[pl.ds(16 * v, 16)] = dstv[pl.ds(16 * v, 16)] * 4
        for v in range(K1 // 16):
            s4 = srcv4[pl.ds(16 * v, 16)]
            d4 = dstv4[pl.ds(16 * v, 16)]
            for h in range(4):
                sv = plsc.load_gather(sflat, [s4 + h])
                dv = plsc.load_gather(dflat, [d4 + h])
                wgrp[pl.ds(80 * h + 16 * v, 16)] = jnp.exp(_leaky(sv + dv))
        for e in range(K1):
            wbuf[e, pl.ds(0, 16)] = plsc.load_gather(wgrp, [zb0 + zbm * e])
        pltpu.sync_copy(wbuf, den_sh.at[dstv], add=True)
        pltpu.sync_copy(wbuf, wtab_hbm.at[c, pl.ds(base, K1)])
        return carry

    lax.fori_loop(0, EPT1 // K1, body, 0)
    plsc.subcore_barrier()
    pltpu.sync_copy(den_sh.at[pl.ds(row0, SLAB)],
                    den_hbm.at[c, pl.ds(row0, SLAB)])


@functools.lru_cache(maxsize=1)
def _edge0():
    return pl.kernel(
        _edge0_body,
        out_type=[
            jax.ShapeDtypeStruct((2, E, 16), jnp.float32),
            jax.ShapeDtypeStruct((2, N, 16), jnp.float32),
        ],
        mesh=plsc.VectorSubcoreMesh(core_axis_name="c", subcore_axis_name="s",
                                    num_cores=NSC, num_subcores=NT),
        compiler_params=pltpu.CompilerParams(needs_layout_passes=False),
        scratch_types=[
            pltpu.VMEM_SHARED((N, 16), jnp.float32),
            pltpu.VMEM((K1,), jnp.int32),
            pltpu.VMEM((K1,), jnp.int32),
            pltpu.VMEM((K1,), jnp.int32),
            pltpu.VMEM((K1,), jnp.int32),
            pltpu.VMEM((K1, 16), jnp.float32),
            pltpu.VMEM((4 * N,), jnp.float32),
            pltpu.VMEM((4 * N,), jnp.float32),
            pltpu.VMEM((336,), jnp.float32),
            pltpu.SemaphoreType.DMA,
        ],
    )


def _edge1_body(src_hbm, dst_hbm, h01_hbm, wtabf_hbm,
                num_hbm,
                num_sh, srcv0, srcv1, srcv2_0, srcv2_1, dstv0, dstv1,
                dstw0, dstw1, hbuf0, hbuf1, wflat0, wflat1,
                sem_in0, sem_in1, sem_g0, sem_g1, sem_s0, sem_s1):
    c = lax.axis_index("c")
    t = lax.axis_index("s")
    cn = c * N
    srcv = (srcv0, srcv1)
    srcv2 = (srcv2_0, srcv2_1)
    dstv = (dstv0, dstv1)
    dstw = (dstw0, dstw1)
    hbuf = (hbuf0, hbuf1)
    wflat = (wflat0, wflat1)
    sem_in = (sem_in0, sem_in1)
    sem_g = (sem_g0, sem_g1)
    sem_s = (sem_s0, sem_s1)

    def zero_body(e, carry):
        for g in range(8):
            hbuf0[e, pl.ds(16 * g, 16)] = jnp.zeros((16,), jnp.float32)
        return carry

    lax.fori_loop(0, K1, zero_body, 0)
    row0 = jnp.minimum(t * SLAB, N - SLAB)
    off = 0
    for nr in ZCHUNKS:
        pltpu.sync_copy(hbuf0.at[pl.ds(0, nr)],
                        num_sh.at[pl.ds(row0 + off, nr)])
        off += nr
    plsc.subcore_barrier()

    nch = EPT1 // K1

    def issue_in(ch, b):
        base = t * EPT1 + ch * K1
        pltpu.async_copy(src_hbm.at[pl.ds(base, K1)], srcv[b], sem_in[b])
        pltpu.async_copy(dst_hbm.at[pl.ds(base, K1)], dstv[b], sem_in[b])
        pltpu.async_copy(wtabf_hbm.at[c, pl.ds(16 * base, 16 * K1)],
                         wflat[b], sem_in[b])

    def wait_in(b):
        pltpu.make_async_copy(src_hbm.at[pl.ds(0, K1)], srcv[b],
                              sem_in[b]).wait()
        pltpu.make_async_copy(dst_hbm.at[pl.ds(0, K1)], dstv[b],
                              sem_in[b]).wait()
        pltpu.make_async_copy(wtabf_hbm.at[c, pl.ds(0, 16 * K1)], wflat[b],
                              sem_in[b]).wait()

    def wait_scatter(b):
        pltpu.make_async_copy(hbuf[b], num_sh.at[dstw[b]], sem_s[b]).wait()

    def issue_gather(b):
        for v in range(K1 // 16):
            srcv2[b][pl.ds(16 * v, 16)] = srcv[b][pl.ds(16 * v, 16)] + cn
        pltpu.async_copy(h01_hbm.at[srcv2[b]], hbuf[b], sem_g[b])

    def proc(b):
        pltpu.make_async_copy(h01_hbm.at[srcv2[b]], hbuf[b], sem_g[b]).wait()
        for v in range(K1 // 16):
            dstw[b][pl.ds(16 * v, 16)] = dstv[b][pl.ds(16 * v, 16)]

        def mule(e, carry):
            for h in range(4):
                ws = plsc.load_gather(
                    wflat[b], [jnp.full((16,), 16 * e + h, jnp.int32)])
                for g in range(2):
                    o = 32 * h + 16 * g
                    hbuf[b][e, pl.ds(o, 16)] = (hbuf[b][e, pl.ds(o, 16)]
                                                * ws)
            return carry

        lax.fori_loop(0, K1, mule, 0, unroll=8)
        pltpu.async_copy(hbuf[b], num_sh.at[dstw[b]], sem_s[b], add=True)

    issue_in(0, 0)
    issue_in(1, 1)
    wait_in(0)
    issue_gather(0)
    wait_in(1)
    issue_gather(1)
    proc(0)
    issue_in(2, 0)
    proc(1)
    issue_in(3, 1)
    wait_in(0)
    wait_scatter(0)
    issue_gather(0)

    def body(i, carry):
        ch = 2 * i
        wait_in(1)
        wait_scatter(1)
        issue_gather(1)
        proc(0)
        issue_in(ch + 2, 0)
        proc(1)
        issue_in(ch + 3, 1)
        wait_in(0)
        wait_scatter(0)
        issue_gather(0)
        return carry

    lax.fori_loop(1, nch // 2 - 1, body, 0)
    wait_in(1)
    wait_scatter(1)
    issue_gather(1)
    proc(0)
    proc(1)
    wait_scatter(0)
    wait_scatter(1)
    plsc.subcore_barrier()
    pltpu.sync_copy(num_sh.at[pl.ds(row0, SLAB)],
                    num_hbm.at[c, pl.ds(row0, SLAB)])


@functools.lru_cache(maxsize=1)
def _edge1():
    return pl.kernel(
        _edge1_body,
        out_type=jax.ShapeDtypeStruct((2, N, 128), jnp.float32),
        mesh=plsc.VectorSubcoreMesh(core_axis_name="c", subcore_axis_name="s",
                                    num_cores=NSC, num_subcores=NT),
        compiler_params=pltpu.CompilerParams(needs_layout_passes=False),
        scratch_types=[
            pltpu.VMEM_SHARED((N, 128), jnp.float32),
            pltpu.VMEM((K1,), jnp.int32),
            pltpu.VMEM((K1,), jnp.int32),
            pltpu.VMEM((K1,), jnp.int32),
            pltpu.VMEM((K1,), jnp.int32),
            pltpu.VMEM((K1,), jnp.int32),
            pltpu.VMEM((K1,), jnp.int32),
            pltpu.VMEM((K1,), jnp.int32),
            pltpu.VMEM((K1,), jnp.int32),
            pltpu.VMEM((K1, 128), jnp.float32),
            pltpu.VMEM((K1, 128), jnp.float32),
            pltpu.VMEM((16 * K1,), jnp.float32),
            pltpu.VMEM((16 * K1,), jnp.float32),
            pltpu.SemaphoreType.DMA,
            pltpu.SemaphoreType.DMA,
            pltpu.SemaphoreType.DMA,
            pltpu.SemaphoreType.DMA,
            pltpu.SemaphoreType.DMA,
            pltpu.SemaphoreType.DMA,
        ],
    )


def _dense2_body(num_ref, den_ref, hh_ref, s_ref, d_ref, b1_ref, w2_ref,
                 a2s_ref, a2d_ref, t2_ref, h2s_ref, s2o_ref, d2o_ref):
    s = s_ref[...][:, :8]
    d = d_ref[...][:, :8]
    wself = jnp.exp(_leaky(s + d))
    den8 = jnp.concatenate([den_ref[0][:, :4], den_ref[1][:, :4]], axis=1)
    dinv = 1.0 / (den8 + wself + 1e-16)
    rowi = lax.broadcasted_iota(jnp.int32, (8, 128), 0)
    coli = lax.broadcasted_iota(jnp.int32, (8, 128), 1) // HID
    h2 = jnp.zeros((t2_ref.shape[0], 16), jnp.float32)
    for c in range(2):
        ec = (coli == (rowi - 4 * c)).astype(jnp.float32)
        wexp = jnp.dot(wself, ec, preferred_element_type=jnp.float32)
        dexp = jnp.dot(dinv, ec, preferred_element_type=jnp.float32)
        out1 = (num_ref[c] + hh_ref[c] * wexp) * dexp \
            + b1_ref[...][:, 128 * c:128 * (c + 1)]
        z = jnp.where(out1 > 0, out1, jnp.exp(out1) - 1.0)
        h2 = h2 + jnp.dot(z, w2_ref[...][128 * c:128 * (c + 1), :],
                          preferred_element_type=jnp.float32)
    ri = lax.broadcasted_iota(jnp.int32, (16, 32), 0)
    ci = lax.broadcasted_iota(jnp.int32, (16, 32), 1)
    aug = (ri == ci).astype(jnp.float32) \
        + a2s_ref[...] * (ci == 16).astype(jnp.float32) \
        + a2d_ref[...] * (ci == 17).astype(jnp.float32)
    t2 = jnp.dot(h2, aug, preferred_element_type=jnp.float32)
    t2_ref[...] = t2
    h2s_ref[0] = h2[:, :8]
    h2s_ref[1] = h2[:, 8:]
    s2o_ref[...] = t2[:, 16:17]
    d2o_ref[...] = t2[:, 17:18]


def _dense2(num, den, hh, s16, d16, b1, W2, a2s, a2d):
    return pl.pallas_call(
        _dense2_body,
        grid=(N // R,),
        in_specs=[
            pl.BlockSpec((2, R, 128), lambda i: (0, i, 0)),
            pl.BlockSpec((2, R, 16), lambda i: (0, i, 0)),
            pl.BlockSpec((2, R, 128), lambda i: (0, i, 0)),
            pl.BlockSpec((R, 16), lambda i: (i, 0)),
            pl.BlockSpec((R, 16), lambda i: (i, 0)),
            pl.BlockSpec((1, 256), lambda i: (0, 0)),
            pl.BlockSpec((256, 16), lambda i: (0, 0)),
            pl.BlockSpec((16, 1), lambda i: (0, 0)),
            pl.BlockSpec((16, 1), lambda i: (0, 0)),
        ],
        out_specs=[
            pl.BlockSpec((R, 32), lambda i: (i, 0)),
            pl.BlockSpec((2, R, 8), lambda i: (0, i, 0)),
            pl.BlockSpec((R, 1), lambda i: (i, 0)),
            pl.BlockSpec((R, 1), lambda i: (i, 0)),
        ],
        out_shape=[
            jax.ShapeDtypeStruct((N, 32), jnp.float32),
            jax.ShapeDtypeStruct((2, N, 8), jnp.float32),
            jax.ShapeDtypeStruct((N, 1), jnp.float32),
            jax.ShapeDtypeStruct((N, 1), jnp.float32),
        ],
    )(num, den, hh, s16, d16, b1, W2, a2s, a2d)


def _edge2_body(src_hbm, dst_hbm, h2s_hbm, s2_hbm, d2_hbm, acc_hbm,
                acc_sh, srcv, dstv, srcv8, mbuf, w2buf, h2pad, s2v, d2v, sem):
    c = lax.axis_index("c")
    t = lax.axis_index("s")

    pltpu.sync_copy(h2s_hbm.at[c], h2pad.at[pl.ds(0, 8 * N)])
    pltpu.sync_copy(s2_hbm, s2v)
    pltpu.sync_copy(d2_hbm, d2v)
    h2pad[pl.ds(8 * N, 16)] = jnp.zeros((16,), jnp.float32)

    def zero_body(e, carry):
        mbuf[e, pl.ds(0, 16)] = jnp.zeros((16,), jnp.float32)
        return carry

    lax.fori_loop(0, K2, zero_body, 0)
    row0 = jnp.minimum(t * SLAB, N - SLAB)
    off = 0
    for nr in ZCHUNKS:
        pltpu.sync_copy(mbuf.at[pl.ds(0, nr)],
                        acc_sh.at[pl.ds(row0 + off, nr)])
        off += nr
    plsc.subcore_barrier()

    lanes = lax.iota(jnp.int32, 16)
    lane8 = lanes < 8
    mask8 = (lanes == 8).astype(jnp.float32)

    def body(j, carry):
        base = t * EPT1 + j * K2
        pltpu.sync_copy(src_hbm.at[pl.ds(base, K2)], srcv)
        pltpu.sync_copy(dst_hbm.at[pl.ds(base, K2)], dstv)
        for v in range(K2 // 16):
            sl = srcv[pl.ds(16 * v, 16)]
            srcv8[pl.ds(16 * v, 16)] = sl * 8
            sv = plsc.load_gather(s2v, [sl])
            dv = plsc.load_gather(d2v, [dstv[pl.ds(16 * v, 16)]])
            w2buf[pl.ds(16 * v, 16)] = jnp.exp(_leaky(sv + dv))
        for e in range(K2):
            esplat = jnp.full((16,), e, jnp.int32)
            bs = plsc.load_gather(srcv8, [esplat])
            idx = jnp.where(lane8, bs + lanes, 8 * N)
            mrow = plsc.load_gather(h2pad, [idx])
            ws = plsc.load_gather(w2buf, [esplat])
            mbuf[e, pl.ds(0, 16)] = (mrow + mask8) * ws
        pltpu.sync_copy(mbuf, acc_sh.at[dstv], add=True)
        return carry

    lax.fori_loop(0, EPT1 // K2, body, 0)
    plsc.subcore_barrier()
    pltpu.sync_copy(acc_sh.at[pl.ds(row0, SLAB)],
                    acc_hbm.at[c, pl.ds(row0, SLAB)])


@functools.lru_cache(maxsize=1)
def _edge2():
    return pl.kernel(
        _edge2_body,
        out_type=jax.ShapeDtypeStruct((2, N, 16), jnp.float32),
        mesh=plsc.VectorSubcoreMesh(core_axis_name="c", subcore_axis_name="s",
                                    num_cores=NSC, num_subcores=NT),
        compiler_params=pltpu.CompilerParams(needs_layout_passes=False),
        scratch_types=[
            pltpu.VMEM_SHARED((N, 16), jnp.float32),
            pltpu.VMEM((K2,), jnp.int32),
            pltpu.VMEM((K2,), jnp.int32),
            pltpu.VMEM((K2,), jnp.int32),
            pltpu.VMEM((K2, 16), jnp.float32),
            pltpu.VMEM((K2,), jnp.float32),
            pltpu.VMEM((8 * N + 16,), jnp.float32),
            pltpu.VMEM((N,), jnp.float32),
            pltpu.VMEM((N,), jnp.float32),
            pltpu.SemaphoreType.DMA,
        ],
    )


def _final_body(acc_ref, t2_ref, b2_ref, out_ref):
    t2 = t2_ref[...]
    h2 = t2[:, :16]
    v = t2[:, 16:17] + t2[:, 17:18]
    w2 = jnp.exp(_leaky(v))
    num2 = jnp.concatenate([acc_ref[0][:, :8], acc_ref[1][:, :8]],
                           axis=1) + h2 * w2
    den2 = acc_ref[0][:, 8:9] + w2 + 1e-16
    o = num2 / den2 + b2_ref[...]
    m = jnp.max(o, axis=1, keepdims=True)
    out_ref[...] = (o - m) - jnp.log(jnp.sum(jnp.exp(o - m), axis=1,
                                             keepdims=True))


def _final(acc, t2, b2):
    return pl.pallas_call(
        _final_body,
        grid=(N // R,),
        in_specs=[
            pl.BlockSpec((2, R, 16), lambda i: (0, i, 0)),
            pl.BlockSpec((R, 32), lambda i: (i, 0)),
            pl.BlockSpec((1, 16), lambda i: (0, 0)),
        ],
        out_specs=pl.BlockSpec((R, 16), lambda i: (i, 0)),
        out_shape=jax.ShapeDtypeStruct((N, 16), jnp.float32),
    )(acc, t2, b2)


def kernel(x, edge_index, W1, a_src1, a_dst1, b1, W2, a_src2, a_dst2, b2):
    src = edge_index[0]
    dst = edge_index[1]
    asf = a_src1.reshape(1, HEADS * HID)
    adf = a_dst1.reshape(1, HEADS * HID)
    hh, s16, d16 = _dense1(x, W1, asf, adf)
    h01 = hh.reshape(2 * N, 128)
    shf = jnp.stack([s16[:, 0:4], s16[:, 4:8]]).reshape(2, 4 * N)
    dhf = jnp.stack([d16[:, 0:4], d16[:, 4:8]]).reshape(2, 4 * N)
    wtab, den = _edge0()(src, dst, shf, dhf)
    num = _edge1()(src, dst, h01, wtab.reshape(2, 16 * E))
    t2, h2s, s2o, d2o = _dense2(num, den, hh, s16, d16, b1.reshape(1, 256),
                                W2, a_src2.reshape(16, 1),
                                a_dst2.reshape(16, 1))
    acc = _edge2()(src, dst, h2s.reshape(2, 8 * N), s2o.reshape(N),
                   d2o.reshape(N))
    return _final(acc, t2, b2.reshape(1, 16))


# speedup vs baseline: 25.4862x; 1.3525x over previous
import functools

import jax
import jax.numpy as jnp
from jax import lax
from jax.experimental import pallas as pl
from jax.experimental.pallas import tpu as pltpu
from jax.experimental.pallas import tpu_sc as plsc

N = 10000
E = 320000
F_IN = 128
HID = 32
HEADS = 8
NUM_CLASS = 16

R = 1000
K1 = 80
K2 = 80
NT = 16
NSC = 2
SLAB = 632
ZCHUNKS = (80, 80, 80, 80, 80, 80, 80, 72)
EPT1 = E // NT
EPT2 = E // (NSC * NT)


def _leaky(v):
    return jnp.where(v > 0, v, 0.2 * v)


def _dense1_body(x_ref, w1_ref, asf_ref, adf_ref, hh_ref, s_ref, d_ref):
    h = jnp.dot(x_ref[...], w1_ref[...], preferred_element_type=jnp.float32)
    row = lax.broadcasted_iota(jnp.int32, (HEADS * HID, 16), 0) // HID
    col = lax.broadcasted_iota(jnp.int32, (HEADS * HID, 16), 1)
    m = (row == col).astype(jnp.float32)
    s_ref[...] = jnp.dot(h * asf_ref[...], m, preferred_element_type=jnp.float32)
    d_ref[...] = jnp.dot(h * adf_ref[...], m, preferred_element_type=jnp.float32)
    hh_ref[0] = h[:, :128]
    hh_ref[1] = h[:, 128:]


def _dense1(x, W1, asf, adf):
    return pl.pallas_call(
        _dense1_body,
        grid=(N // R,),
        in_specs=[
            pl.BlockSpec((R, F_IN), lambda i: (i, 0)),
            pl.BlockSpec((F_IN, HEADS * HID), lambda i: (0, 0)),
            pl.BlockSpec((1, HEADS * HID), lambda i: (0, 0)),
            pl.BlockSpec((1, HEADS * HID), lambda i: (0, 0)),
        ],
        out_specs=[
            pl.BlockSpec((2, R, 128), lambda i: (0, i, 0)),
            pl.BlockSpec((R, 16), lambda i: (i, 0)),
            pl.BlockSpec((R, 16), lambda i: (i, 0)),
        ],
        out_shape=[
            jax.ShapeDtypeStruct((2, N, 128), jnp.float32),
            jax.ShapeDtypeStruct((N, 16), jnp.float32),
            jax.ShapeDtypeStruct((N, 16), jnp.float32),
        ],
    )(x, W1, asf, adf)


def _edge0_body(src_hbm, dst_hbm, sh_hbm, dh_hbm,
                wtab_hbm, den_hbm,
                den_sh, srcv, srcv4, dstv, dstv4, wbuf, sflat, dflat,
                wgrp, sem):
    c = lax.axis_index("c")
    t = lax.axis_index("s")

    pltpu.sync_copy(sh_hbm.at[c], sflat)
    pltpu.sync_copy(dh_hbm.at[c], dflat)

    def zero_body(e, carry):
        wbuf[e, pl.ds(0, 16)] = jnp.zeros((16,), jnp.float32)
        return carry

    lax.fori_loop(0, K1, zero_body, 0)
    for g in range(21):
        wgrp[pl.ds(16 * g, 16)] = jnp.zeros((16,), jnp.float32)
    row0 = jnp.minimum(t * SLAB, N - SLAB)
    off = 0
    for nr in ZCHUNKS:
        pltpu.sync_copy(wbuf.at[pl.ds(0, nr)],
                        den_sh.at[pl.ds(row0 + off, nr)])
        off += nr
    plsc.subcore_barrier()

    lanes = lax.iota(jnp.int32, 16)
    zb0 = jnp.where(lanes < 4, 80 * lanes, 320)
    zbm = (lanes < 4).astype(jnp.int32)

    def body(j, carry):
        base = t * EPT1 + j * K1
        pltpu.sync_copy(src_hbm.at[pl.ds(base, K1)], srcv)
        pltpu.sync_copy(dst_hbm.at[pl.ds(base, K1)], dstv)
        for v in range(K1 // 16):
            srcv4[pl.ds(16 * v, 16)] = srcv[pl.ds(16 * v, 16)] * 4
            dstv4[pl.ds(16 * v, 16)] = dstv[pl.ds(16 * v, 16)] * 4
        for v in range(K1 // 16):
            s4 = srcv4[pl.ds(16 * v, 16)]
            d4 = dstv4[pl.ds(16 * v, 16)]
            for h in range(4):
                sv = plsc.load_gather(sflat, [s4 + h])
                dv = plsc.load_gather(dflat, [d4 + h])
                wgrp[pl.ds(80 * h + 16 * v, 16)] = jnp.exp(_leaky(sv + dv))
        for e in range(K1):
            wbuf[e, pl.ds(0, 16)] = plsc.load_gather(wgrp, [zb0 + zbm * e])
        pltpu.sync_copy(wbuf, den_sh.at[dstv], add=True)
        pltpu.sync_copy(wbuf, wtab_hbm.at[c, pl.ds(base, K1)])
        return carry

    lax.fori_loop(0, EPT1 // K1, body, 0)
    plsc.subcore_barrier()
    pltpu.sync_copy(den_sh.at[pl.ds(row0, SLAB)],
                    den_hbm.at[c, pl.ds(row0, SLAB)])


@functools.lru_cache(maxsize=1)
def _edge0():
    return pl.kernel(
        _edge0_body,
        out_type=[
            jax.ShapeDtypeStruct((2, E, 16), jnp.float32),
            jax.ShapeDtypeStruct((2, N, 16), jnp.float32),
        ],
        mesh=plsc.VectorSubcoreMesh(core_axis_name="c", subcore_axis_name="s",
                                    num_cores=NSC, num_subcores=NT),
        compiler_params=pltpu.CompilerParams(needs_layout_passes=False),
        scratch_types=[
            pltpu.VMEM_SHARED((N, 16), jnp.float32),
            pltpu.VMEM((K1,), jnp.int32),
            pltpu.VMEM((K1,), jnp.int32),
            pltpu.VMEM((K1,), jnp.int32),
            pltpu.VMEM((K1,), jnp.int32),
            pltpu.VMEM((K1, 16), jnp.float32),
            pltpu.VMEM((4 * N,), jnp.float32),
            pltpu.VMEM((4 * N,), jnp.float32),
            pltpu.VMEM((336,), jnp.float32),
            pltpu.SemaphoreType.DMA,
        ],
    )


def _edge1_body(src_hbm, dst_hbm, h01_hbm, wtabf_hbm,
                num_hbm,
                num_sh, srcv0, srcv1, srcv2_0, srcv2_1, dstv0, dstv1,
                dstw0, dstw1, hbuf0, hbuf1, wflat0, wflat1, wfl1d,
                sem_in0, sem_in1, sem_g0, sem_g1, sem_s0, sem_s1):
    c = lax.axis_index("c")
    t = lax.axis_index("s")
    cn = c * N
    srcv = (srcv0, srcv1)
    srcv2 = (srcv2_0, srcv2_1)
    dstv = (dstv0, dstv1)
    dstw = (dstw0, dstw1)
    hbuf = (hbuf0, hbuf1)
    wflat = (wflat0, wflat1)
    sem_in = (sem_in0, sem_in1)
    sem_g = (sem_g0, sem_g1)
    sem_s = (sem_s0, sem_s1)

    def zero_body(e, carry):
        for g in range(8):
            hbuf0[e, pl.ds(16 * g, 16)] = jnp.zeros((16,), jnp.float32)
        return carry

    lax.fori_loop(0, K1, zero_body, 0)
    row0 = jnp.minimum(t * SLAB, N - SLAB)
    off = 0
    for nr in ZCHUNKS:
        pltpu.sync_copy(hbuf0.at[pl.ds(0, nr)],
                        num_sh.at[pl.ds(row0 + off, nr)])
        off += nr
    plsc.subcore_barrier()

    nch = EPT1 // K1

    def issue_in(ch, b):
        base = t * EPT1 + ch * K1
        pltpu.async_copy(src_hbm.at[pl.ds(base, K1)], srcv[b], sem_in[b])
        pltpu.async_copy(dst_hbm.at[pl.ds(base, K1)], dstv[b], sem_in[b])

    def load_w(ch, b):
        base = t * EPT1 + ch * K1
        pltpu.sync_copy(wtabf_hbm.at[c, pl.ds(base, K1)], wflat[b])

    def wait_in(b):
        pltpu.make_async_copy(src_hbm.at[pl.ds(0, K1)], srcv[b],
                              sem_in[b]).wait()
        pltpu.make_async_copy(dst_hbm.at[pl.ds(0, K1)], dstv[b],
                              sem_in[b]).wait()

    def wait_scatter(b):
        pltpu.make_async_copy(hbuf[b], num_sh.at[dstw[b]], sem_s[b]).wait()

    def issue_gather(b):
        for v in range(K1 // 16):
            srcv2[b][pl.ds(16 * v, 16)] = srcv[b][pl.ds(16 * v, 16)] + cn
        pltpu.async_copy(h01_hbm.at[srcv2[b]], hbuf[b], sem_g[b])

    def proc(ch, b):
        load_w(ch, b)
        pltpu.make_async_copy(h01_hbm.at[srcv2[b]], hbuf[b], sem_g[b]).wait()
        for v in range(K1 // 16):
            dstw[b][pl.ds(16 * v, 16)] = dstv[b][pl.ds(16 * v, 16)]

        def flat(e, carry):
            wfl1d[pl.ds(16 * e, 16)] = wflat[b][e, pl.ds(0, 16)]
            return carry

        lax.fori_loop(0, K1, flat, 0, unroll=8)

        def mule(e, carry):
            for h in range(4):
                ws = plsc.load_gather(
                    wfl1d, [jnp.full((16,), 16 * e + h, jnp.int32)])
                for g in range(2):
                    o = 32 * h + 16 * g
                    hbuf[b][e, pl.ds(o, 16)] = (hbuf[b][e, pl.ds(o, 16)]
                                                * ws)
            return carry

        lax.fori_loop(0, K1, mule, 0, unroll=8)
        pltpu.async_copy(hbuf[b], num_sh.at[dstw[b]], sem_s[b], add=True)

    issue_in(0, 0)
    issue_in(1, 1)
    wait_in(0)
    issue_gather(0)
    wait_in(1)
    issue_gather(1)
    proc(0, 0)
    issue_in(2, 0)
    proc(1, 1)
    issue_in(3, 1)
    wait_in(0)
    wait_scatter(0)
    issue_gather(0)

    def body(i, carry):
        ch = 2 * i
        wait_in(1)
        wait_scatter(1)
        issue_gather(1)
        proc(ch, 0)
        issue_in(ch + 2, 0)
        proc(ch + 1, 1)
        issue_in(ch + 3, 1)
        wait_in(0)
        wait_scatter(0)
        issue_gather(0)
        return carry

    lax.fori_loop(1, nch // 2 - 1, body, 0)
    wait_in(1)
    wait_scatter(1)
    issue_gather(1)
    proc(nch - 2, 0)
    proc(nch - 1, 1)
    wait_scatter(0)
    wait_scatter(1)
    plsc.subcore_barrier()
    pltpu.sync_copy(num_sh.at[pl.ds(row0, SLAB)],
                    num_hbm.at[c, pl.ds(row0, SLAB)])


@functools.lru_cache(maxsize=1)
def _edge1():
    return pl.kernel(
        _edge1_body,
        out_type=jax.ShapeDtypeStruct((2, N, 128), jnp.float32),
        mesh=plsc.VectorSubcoreMesh(core_axis_name="c", subcore_axis_name="s",
                                    num_cores=NSC, num_subcores=NT),
        compiler_params=pltpu.CompilerParams(needs_layout_passes=False),
        scratch_types=[
            pltpu.VMEM_SHARED((N, 128), jnp.float32),
            pltpu.VMEM((K1,), jnp.int32),
            pltpu.VMEM((K1,), jnp.int32),
            pltpu.VMEM((K1,), jnp.int32),
            pltpu.VMEM((K1,), jnp.int32),
            pltpu.VMEM((K1,), jnp.int32),
            pltpu.VMEM((K1,), jnp.int32),
            pltpu.VMEM((K1,), jnp.int32),
            pltpu.VMEM((K1,), jnp.int32),
            pltpu.VMEM((K1, 128), jnp.float32),
            pltpu.VMEM((K1, 128), jnp.float32),
            pltpu.VMEM((K1, 16), jnp.float32),
            pltpu.VMEM((K1, 16), jnp.float32),
            pltpu.VMEM((16 * K1,), jnp.float32),
            pltpu.SemaphoreType.DMA,
            pltpu.SemaphoreType.DMA,
            pltpu.SemaphoreType.DMA,
            pltpu.SemaphoreType.DMA,
            pltpu.SemaphoreType.DMA,
            pltpu.SemaphoreType.DMA,
        ],
    )


def _dense2_body(num_ref, den_ref, hh_ref, s_ref, d_ref, b1_ref, w2_ref,
                 a2s_ref, a2d_ref, t2_ref, h2s_ref, s2o_ref, d2o_ref):
    s = s_ref[...][:, :8]
    d = d_ref[...][:, :8]
    wself = jnp.exp(_leaky(s + d))
    den8 = jnp.concatenate([den_ref[0][:, :4], den_ref[1][:, :4]], axis=1)
    dinv = 1.0 / (den8 + wself + 1e-16)
    rowi = lax.broadcasted_iota(jnp.int32, (8, 128), 0)
    coli = lax.broadcasted_iota(jnp.int32, (8, 128), 1) // HID
    h2 = jnp.zeros((t2_ref.shape[0], 16), jnp.float32)
    for c in range(2):
        ec = (coli == (rowi - 4 * c)).astype(jnp.float32)
        wexp = jnp.dot(wself, ec, preferred_element_type=jnp.float32)
        dexp = jnp.dot(dinv, ec, preferred_element_type=jnp.float32)
        out1 = (num_ref[c] + hh_ref[c] * wexp) * dexp \
            + b1_ref[...][:, 128 * c:128 * (c + 1)]
        z = jnp.where(out1 > 0, out1, jnp.exp(out1) - 1.0)
        h2 = h2 + jnp.dot(z, w2_ref[...][128 * c:128 * (c + 1), :],
                          preferred_element_type=jnp.float32)
    ri = lax.broadcasted_iota(jnp.int32, (16, 32), 0)
    ci = lax.broadcasted_iota(jnp.int32, (16, 32), 1)
    aug = (ri == ci).astype(jnp.float32) \
        + a2s_ref[...] * (ci == 16).astype(jnp.float32) \
        + a2d_ref[...] * (ci == 17).astype(jnp.float32)
    t2 = jnp.dot(h2, aug, preferred_element_type=jnp.float32)
    t2_ref[...] = t2
    h2s_ref[0] = h2[:, :8]
    h2s_ref[1] = h2[:, 8:]
    s2o_ref[...] = t2[:, 16:17]
    d2o_ref[...] = t2[:, 17:18]


def _dense2(num, den, hh, s16, d16, b1, W2, a2s, a2d):
    return pl.pallas_call(
        _dense2_body,
        grid=(N // R,),
        in_specs=[
            pl.BlockSpec((2, R, 128), lambda i: (0, i, 0)),
            pl.BlockSpec((2, R, 16), lambda i: (0, i, 0)),
            pl.BlockSpec((2, R, 128), lambda i: (0, i, 0)),
            pl.BlockSpec((R, 16), lambda i: (i, 0)),
            pl.BlockSpec((R, 16), lambda i: (i, 0)),
            pl.BlockSpec((1, 256), lambda i: (0, 0)),
            pl.BlockSpec((256, 16), lambda i: (0, 0)),
            pl.BlockSpec((16, 1), lambda i: (0, 0)),
            pl.BlockSpec((16, 1), lambda i: (0, 0)),
        ],
        out_specs=[
            pl.BlockSpec((R, 32), lambda i: (i, 0)),
            pl.BlockSpec((2, R, 8), lambda i: (0, i, 0)),
            pl.BlockSpec((R, 1), lambda i: (i, 0)),
            pl.BlockSpec((R, 1), lambda i: (i, 0)),
        ],
        out_shape=[
            jax.ShapeDtypeStruct((N, 32), jnp.float32),
            jax.ShapeDtypeStruct((2, N, 8), jnp.float32),
            jax.ShapeDtypeStruct((N, 1), jnp.float32),
            jax.ShapeDtypeStruct((N, 1), jnp.float32),
        ],
    )(num, den, hh, s16, d16, b1, W2, a2s, a2d)


def _edge2_body(src_hbm, dst_hbm, h2s_hbm, s2_hbm, d2_hbm, acc_hbm,
                acc_sh, srcv, dstv, srcv8, mbuf, w2buf, h2pad, s2v, d2v, sem):
    c = lax.axis_index("c")
    t = lax.axis_index("s")

    pltpu.sync_copy(h2s_hbm.at[c], h2pad.at[pl.ds(0, 8 * N)])
    pltpu.sync_copy(s2_hbm, s2v)
    pltpu.sync_copy(d2_hbm, d2v)
    h2pad[pl.ds(8 * N, 16)] = jnp.zeros((16,), jnp.float32)

    def zero_body(e, carry):
        mbuf[e, pl.ds(0, 16)] = jnp.zeros((16,), jnp.float32)
        return carry

    lax.fori_loop(0, K2, zero_body, 0)
    row0 = jnp.minimum(t * SLAB, N - SLAB)
    off = 0
    for nr in ZCHUNKS:
        pltpu.sync_copy(mbuf.at[pl.ds(0, nr)],
                        acc_sh.at[pl.ds(row0 + off, nr)])
        off += nr
    plsc.subcore_barrier()

    lanes = lax.iota(jnp.int32, 16)
    lane8 = lanes < 8
    mask8 = (lanes == 8).astype(jnp.float32)

    def body(j, carry):
        base = t * EPT1 + j * K2
        pltpu.sync_copy(src_hbm.at[pl.ds(base, K2)], srcv)
        pltpu.sync_copy(dst_hbm.at[pl.ds(base, K2)], dstv)
        for v in range(K2 // 16):
            sl = srcv[pl.ds(16 * v, 16)]
            srcv8[pl.ds(16 * v, 16)] = sl * 8
            sv = plsc.load_gather(s2v, [sl])
            dv = plsc.load_gather(d2v, [dstv[pl.ds(16 * v, 16)]])
            w2buf[pl.ds(16 * v, 16)] = jnp.exp(_leaky(sv + dv))
        for e in range(K2):
            esplat = jnp.full((16,), e, jnp.int32)
            bs = plsc.load_gather(srcv8, [esplat])
            idx = jnp.where(lane8, bs + lanes, 8 * N)
            mrow = plsc.load_gather(h2pad, [idx])
            ws = plsc.load_gather(w2buf, [esplat])
            mbuf[e, pl.ds(0, 16)] = (mrow + mask8) * ws
        pltpu.sync_copy(mbuf, acc_sh.at[dstv], add=True)
        return carry

    lax.fori_loop(0, EPT1 // K2, body, 0)
    plsc.subcore_barrier()
    pltpu.sync_copy(acc_sh.at[pl.ds(row0, SLAB)],
                    acc_hbm.at[c, pl.ds(row0, SLAB)])


@functools.lru_cache(maxsize=1)
def _edge2():
    return pl.kernel(
        _edge2_body,
        out_type=jax.ShapeDtypeStruct((2, N, 16), jnp.float32),
        mesh=plsc.VectorSubcoreMesh(core_axis_name="c", subcore_axis_name="s",
                                    num_cores=NSC, num_subcores=NT),
        compiler_params=pltpu.CompilerParams(needs_layout_passes=False),
        scratch_types=[
            pltpu.VMEM_SHARED((N, 16), jnp.float32),
            pltpu.VMEM((K2,), jnp.int32),
            pltpu.VMEM((K2,), jnp.int32),
            pltpu.VMEM((K2,), jnp.int32),
            pltpu.VMEM((K2, 16), jnp.float32),
            pltpu.VMEM((K2,), jnp.float32),
            pltpu.VMEM((8 * N + 16,), jnp.float32),
            pltpu.VMEM((N,), jnp.float32),
            pltpu.VMEM((N,), jnp.float32),
            pltpu.SemaphoreType.DMA,
        ],
    )


def _final_body(acc_ref, t2_ref, b2_ref, out_ref):
    t2 = t2_ref[...]
    h2 = t2[:, :16]
    v = t2[:, 16:17] + t2[:, 17:18]
    w2 = jnp.exp(_leaky(v))
    num2 = jnp.concatenate([acc_ref[0][:, :8], acc_ref[1][:, :8]],
                           axis=1) + h2 * w2
    den2 = acc_ref[0][:, 8:9] + w2 + 1e-16
    o = num2 / den2 + b2_ref[...]
    m = jnp.max(o, axis=1, keepdims=True)
    out_ref[...] = (o - m) - jnp.log(jnp.sum(jnp.exp(o - m), axis=1,
                                             keepdims=True))


def _final(acc, t2, b2):
    return pl.pallas_call(
        _final_body,
        grid=(N // R,),
        in_specs=[
            pl.BlockSpec((2, R, 16), lambda i: (0, i, 0)),
            pl.BlockSpec((R, 32), lambda i: (i, 0)),
            pl.BlockSpec((1, 16), lambda i: (0, 0)),
        ],
        out_specs=pl.BlockSpec((R, 16), lambda i: (i, 0)),
        out_shape=jax.ShapeDtypeStruct((N, 16), jnp.float32),
    )(acc, t2, b2)


def kernel(x, edge_index, W1, a_src1, a_dst1, b1, W2, a_src2, a_dst2, b2):
    src = edge_index[0]
    dst = edge_index[1]
    asf = a_src1.reshape(1, HEADS * HID)
    adf = a_dst1.reshape(1, HEADS * HID)
    hh, s16, d16 = _dense1(x, W1, asf, adf)
    h01 = hh.reshape(2 * N, 128)
    shf = jnp.stack([s16[:, 0:4], s16[:, 4:8]]).reshape(2, 4 * N)
    dhf = jnp.stack([d16[:, 0:4], d16[:, 4:8]]).reshape(2, 4 * N)
    wtab, den = _edge0()(src, dst, shf, dhf)
    num = _edge1()(src, dst, h01, wtab)
    t2, h2s, s2o, d2o = _dense2(num, den, hh, s16, d16, b1.reshape(1, 256),
                                W2, a_src2.reshape(16, 1),
                                a_dst2.reshape(16, 1))
    acc = _edge2()(src, dst, h2s.reshape(2, 8 * N), s2o.reshape(N),
                   d2o.reshape(N))
    return _final(acc, t2, b2.reshape(1, 16))
